# Initial kernel scaffold; baseline (speedup 1.0000x reference)
import jax
import jax.numpy as jnp
from jax.experimental import pallas as pl


def kernel(adj, x, enc_mask_token):
    raise NotImplementedError("write your pallas kernel here")


# speedup vs baseline: 5.0982x; 5.0982x over previous
import functools

import jax
import jax.numpy as jnp
import numpy as np
from jax import lax
from jax.experimental import pallas as pl
from jax.experimental.pallas import tpu as pltpu
from jax.experimental.pallas import tpu_sc as plsc

_MASK_RATE = 0.5
_REPLACE_RATE = 0.05
_MASK_TOKEN_RATE = 1.0 - _REPLACE_RATE

_NC = 2
_NS = 16
_NW = _NC * _NS
_C = 128


@functools.lru_cache(maxsize=None)
def _plan(num_nodes: int):
    num_mask = int(_MASK_RATE * num_nodes)
    cpu = jax.local_devices(backend="cpu")[0]
    with jax.ensure_compile_time_eval(), jax.default_device(cpu):
        key = jax.random.key(42)
        kp, km, kn = jax.random.split(key, 3)
        perm = np.asarray(jax.random.permutation(kp, num_nodes))
        perm_mask = np.asarray(jax.random.permutation(km, num_mask))
        noise_all = np.asarray(jax.random.permutation(kn, num_nodes))
    mask_nodes = perm[:num_mask]
    keep_nodes = perm[num_mask:]
    num_noise = int(_REPLACE_RATE * num_mask)
    num_token = int(_MASK_TOKEN_RATE * num_mask)
    token_nodes = mask_nodes[perm_mask[:num_token]]
    noise_nodes = mask_nodes[perm_mask[num_mask - num_noise:]]
    noise_src = noise_all[:num_noise]

    assert np.intersect1d(token_nodes, noise_nodes).size == 0

    gather_src = np.arange(num_nodes, dtype=np.int32)
    gather_src[noise_nodes] = noise_src.astype(np.int32)

    is_token = np.zeros(num_nodes, dtype=bool)
    is_token[token_nodes] = True
    nt_dst = np.nonzero(~is_token)[0].astype(np.int32)
    nt_src = gather_src[nt_dst]
    tk_dst = np.sort(token_nodes).astype(np.int32)

    def pad_chunks(a, pad_val):
        per = _NW * _C
        chunks = -(-a.size // per)
        out = np.full(chunks * per, pad_val, dtype=np.int32)
        out[: a.size] = a
        return out.reshape(_NW, chunks, _C), chunks

    nt_dst3, cha = pad_chunks(nt_dst, nt_dst[0])
    nt_src3, _ = pad_chunks(nt_src, nt_src[0])
    tk_dst3, chb = pad_chunks(tk_dst, tk_dst[0])

    return dict(
        mask_nodes=mask_nodes.astype(np.int32),
        keep_nodes=keep_nodes.astype(np.int32),
        nt_dst3=nt_dst3, nt_src3=nt_src3, tk_dst3=tk_dst3,
        cha=cha, chb=chb,
    )


def _sc_remap(x, enc_mask_token, nts, ntd, tkd, cha, chb):
    num_nodes, d = x.shape
    mesh = plsc.VectorSubcoreMesh(core_axis_name="c", subcore_axis_name="s")

    @functools.partial(
        pl.kernel,
        out_type=jax.ShapeDtypeStruct((num_nodes, d), x.dtype),
        mesh=mesh,
        scratch_types=[
            pltpu.VMEM((_C,), jnp.int32),
            pltpu.VMEM((_C,), jnp.int32),
            pltpu.VMEM((_C,), jnp.int32),
            pltpu.VMEM((_C, d), jnp.float32),
            pltpu.VMEM((_C, d), jnp.float32),
            pltpu.SemaphoreType.DMA,
            pltpu.SemaphoreType.DMA,
        ],
    )
    def k(x_hbm, tok_hbm, nts_hbm, ntd_hbm, tkd_hbm, out_hbm,
          idxs_v, idxd_v, zidx_v, rows_v, fill_v, gsem, ssem):
        wid = lax.axis_index("s") * _NC + lax.axis_index("c")

        def step_a(i, carry):
            pltpu.sync_copy(nts_hbm.at[wid, i], idxs_v)
            pltpu.sync_copy(ntd_hbm.at[wid, i], idxd_v)
            pltpu.async_copy(x_hbm.at[idxs_v], rows_v, gsem).wait()
            pltpu.async_copy(rows_v, out_hbm.at[idxd_v], ssem).wait()
            return carry

        lax.fori_loop(0, cha, step_a, 0, unroll=False)

        for j in range(_C // 16):
            zidx_v[pl.ds(j * 16, 16)] = jnp.zeros((16,), jnp.int32)
        pltpu.async_copy(tok_hbm.at[zidx_v], fill_v, gsem).wait()

        def step_b(i, carry):
            pltpu.sync_copy(tkd_hbm.at[wid, i], idxd_v)
            pltpu.async_copy(fill_v, out_hbm.at[idxd_v], ssem).wait()
            return carry

        lax.fori_loop(0, chb, step_b, 0, unroll=False)

    return k(x, enc_mask_token, nts, ntd, tkd)


def kernel(adj, x, enc_mask_token):
    p = _plan(x.shape[0])
    out_x = _sc_remap(
        x, enc_mask_token,
        jnp.asarray(p["nt_src3"]), jnp.asarray(p["nt_dst3"]),
        jnp.asarray(p["tk_dst3"]), p["cha"], p["chb"],
    )
    return (adj, out_x, jnp.asarray(p["mask_nodes"]), jnp.asarray(p["keep_nodes"]))
